# baseline (speedup 1.0000x reference)
import jax
import jax.numpy as jnp
from jax.experimental import pallas as pl
from jax.experimental.pallas import tpu as pltpu

_N, _F, _T = 32, 360, 2048
_SLOTS = 6


def _ce_manual(x_hbm, oh_hbm, acc_ref, bufx, bufoh, semx, semoh):
    def start(j):
        s = jax.lax.rem(j, _SLOTS)
        pltpu.make_async_copy(x_hbm.at[j], bufx.at[s], semx.at[s]).start()
        pltpu.make_async_copy(oh_hbm.at[j], bufoh.at[s], semoh.at[s]).start()

    for j in range(_SLOTS):
        start(j)

    def body(j, total):
        s = jax.lax.rem(j, _SLOTS)
        pltpu.make_async_copy(x_hbm.at[j], bufx.at[s], semx.at[s]).wait()
        pltpu.make_async_copy(oh_hbm.at[j], bufoh.at[s], semoh.at[s]).wait()

        x = bufx[s]
        oh = bufoh[s]
        l = jnp.log(x + 1e-20)
        s_prod = jnp.sum(oh * l)
        colsum_l = jnp.sum(l, axis=0)
        zero_col = jnp.max(jnp.abs(oh), axis=0) == 0.0
        corr = jnp.sum(jnp.where(zero_col, colsum_l, 0.0))

        @pl.when(j + _SLOTS < _N)
        def _():
            start(j + _SLOTS)

        return total + s_prod + corr * (1.0 / _F)

    total = jax.lax.fori_loop(0, _N, body, jnp.float32(0.0))
    acc_ref[0, 0] = total * (-1.0 / (_T * _N))


def kernel(output, one_hot):
    out = jnp.reshape(output, (_N, _F, _T))
    acc = pl.pallas_call(
        _ce_manual,
        in_specs=[
            pl.BlockSpec(memory_space=pltpu.MemorySpace.HBM),
            pl.BlockSpec(memory_space=pltpu.MemorySpace.HBM),
        ],
        out_specs=pl.BlockSpec(memory_space=pltpu.SMEM),
        out_shape=jax.ShapeDtypeStruct((1, 1), jnp.float32),
        scratch_shapes=[
            pltpu.VMEM((_SLOTS, _F, _T), jnp.float32),
            pltpu.VMEM((_SLOTS, _F, _T), jnp.float32),
            pltpu.SemaphoreType.DMA((_SLOTS,)),
            pltpu.SemaphoreType.DMA((_SLOTS,)),
        ],
    )(out, one_hot)
    return jnp.reshape(acc, ())


# speedup vs baseline: 1.0049x; 1.0049x over previous
import jax
import jax.numpy as jnp
from jax.experimental import pallas as pl
from jax.experimental.pallas import tpu as pltpu

_N, _F, _T = 32, 360, 2048
_SLOTS = 4


def _ce_manual(x_hbm, oh_hbm, acc_ref, bufx, bufoh, semx, semoh):
    def start(j):
        s = jax.lax.rem(j, _SLOTS)
        pltpu.make_async_copy(x_hbm.at[j], bufx.at[s], semx.at[s]).start()
        pltpu.make_async_copy(oh_hbm.at[j], bufoh.at[s], semoh.at[s]).start()

    for j in range(_SLOTS):
        start(j)

    def body(j, total):
        s = jax.lax.rem(j, _SLOTS)
        pltpu.make_async_copy(x_hbm.at[j], bufx.at[s], semx.at[s]).wait()
        pltpu.make_async_copy(oh_hbm.at[j], bufoh.at[s], semoh.at[s]).wait()

        x = bufx[s]
        oh = bufoh[s]
        l = jnp.log(x + 1e-20)
        s_prod = jnp.sum(oh * l)
        colsum_l = jnp.sum(l, axis=0)
        zero_col = jnp.max(jnp.abs(oh), axis=0) == 0.0
        corr = jnp.sum(jnp.where(zero_col, colsum_l, 0.0))

        @pl.when(j + _SLOTS < _N)
        def _():
            start(j + _SLOTS)

        return total + s_prod + corr * (1.0 / _F)

    total = jax.lax.fori_loop(0, _N, body, jnp.float32(0.0))
    acc_ref[0, 0] = total * (-1.0 / (_T * _N))


def kernel(output, one_hot):
    out = jnp.reshape(output, (_N, _F, _T))
    acc = pl.pallas_call(
        _ce_manual,
        in_specs=[
            pl.BlockSpec(memory_space=pltpu.MemorySpace.HBM),
            pl.BlockSpec(memory_space=pltpu.MemorySpace.HBM),
        ],
        out_specs=pl.BlockSpec(memory_space=pltpu.SMEM),
        out_shape=jax.ShapeDtypeStruct((1, 1), jnp.float32),
        scratch_shapes=[
            pltpu.VMEM((_SLOTS, _F, _T), jnp.float32),
            pltpu.VMEM((_SLOTS, _F, _T), jnp.float32),
            pltpu.SemaphoreType.DMA((_SLOTS,)),
            pltpu.SemaphoreType.DMA((_SLOTS,)),
        ],
    )(out, one_hot)
    return jnp.reshape(acc, ())
